# baseline (speedup 1.0000x reference)
import functools

import jax
import jax.numpy as jnp
from jax.experimental import pallas as pl
from jax.experimental.pallas import tpu as pltpu


def _pick_block(n, candidates):
    for b in candidates:
        if n % b == 0:
            return b
    return n


def _pass1_kernel(h_ref, w_ref, bl_ref, a_ref, out_ref):
    c, bn, f = h_ref.shape
    x = h_ref[...].reshape(c * bn, f)
    s = jnp.tanh(
        jnp.dot(x, w_ref[...], preferred_element_type=jnp.float32) + bl_ref[...]
    )
    t = jnp.dot(s, a_ref[...], preferred_element_type=jnp.float32)
    out_ref[0, 0, :] = jnp.sum(t.reshape(c, bn), axis=1)


def _pass2_kernel(bp_ref, h_ref, out_ref, *, inv_confs):
    c = h_ref.shape[0]
    b = jnp.sum(bp_ref[...], axis=0).reshape(c) * inv_confs
    w = jax.nn.softmax(b)
    hb = h_ref[...]
    acc = hb[0] * w[0]
    for i in range(1, c):
        acc = acc + hb[i] * w[i]
    out_ref[...] = acc


def kernel(h, W, b_lin, attn_vector, num_confs):
    del num_confs
    c, n, f = h.shape
    fo = W.shape[1]

    bn1 = _pick_block(n, (1000, 2000, 800, 512, 400, 250, 200, 128, 100, 80,
                          50, 40, 25, 20, 16, 10, 8, 5, 4, 2, 1))
    nb1 = n // bn1
    partials = pl.pallas_call(
        _pass1_kernel,
        grid=(nb1,),
        in_specs=[
            pl.BlockSpec((c, bn1, f), lambda i: (0, i, 0)),
            pl.BlockSpec((f, fo), lambda i: (0, 0)),
            pl.BlockSpec((1, fo), lambda i: (0, 0)),
            pl.BlockSpec((fo, 1), lambda i: (0, 0)),
        ],
        out_specs=pl.BlockSpec((1, 1, c), lambda i: (i, 0, 0)),
        out_shape=jax.ShapeDtypeStruct((nb1, 1, c), jnp.float32),
        compiler_params=pltpu.CompilerParams(
            dimension_semantics=("arbitrary",),
        ),
    )(h, W, b_lin.reshape(1, fo), attn_vector.reshape(fo, 1))

    bn2 = _pick_block(n, (2000, 1000, 800, 512, 400, 250, 200, 128, 100, 80,
                          50, 40, 25, 20, 16, 10, 8, 5, 4, 2, 1))
    nb2 = n // bn2
    out = pl.pallas_call(
        functools.partial(_pass2_kernel, inv_confs=1.0 / c),
        grid=(nb2,),
        in_specs=[
            pl.BlockSpec((nb1, 1, c), lambda i: (0, 0, 0)),
            pl.BlockSpec((c, bn2, f), lambda i: (0, i, 0)),
        ],
        out_specs=pl.BlockSpec((bn2, f), lambda i: (i, 0)),
        out_shape=jax.ShapeDtypeStruct((n, f), jnp.float32),
        compiler_params=pltpu.CompilerParams(
            dimension_semantics=("arbitrary",),
        ),
    )(partials, h)
    return out
